# initial kernel scaffold (unmeasured)
import jax
import jax.numpy as jnp
from jax import lax
from jax.experimental import pallas as pl
from jax.experimental.pallas import tpu as pltpu

N_DEV = 16
SQ = 1024
D_MODEL = 1024
H_PER = 8
DH = 128
BLK = 64
CHUNK = SQ // N_DEV
SCALE = 0.08838834764831843


def kernel(x, Wq, K_ext, V_ext, Wo):
    i = lax.axis_index("i")
    x2 = x[0]
    K = lax.dynamic_slice_in_dim(K_ext[0], i * H_PER, H_PER, axis=1)
    V = lax.dynamic_slice_in_dim(V_ext[0], i * H_PER, H_PER, axis=1)
    K = jnp.transpose(K, (1, 0, 2))
    V = jnp.transpose(V, (1, 0, 2))

    def body(x_ref, wq_ref, k_ref, v_ref, wo_ref, out_ref,
             recv_buf, rs_send_sems, rs_recv_sems,
             ag_send_sems, ag_recv_sems, rs_credit, ag_credit):
        my = lax.axis_index("i")
        left = lax.rem(my + N_DEV - 1, N_DEV)
        right = lax.rem(my + 1, N_DEV)

        barrier = pltpu.get_barrier_semaphore()
        for nbr in (left, right):
            pl.semaphore_signal(barrier, inc=1, device_id=(nbr,),
                                device_id_type=pl.DeviceIdType.MESH)
        pl.semaphore_wait(barrier, 2)

        Q = jnp.dot(x_ref[:], wq_ref[:], preferred_element_type=jnp.float32)
        rb = lax.broadcasted_iota(jnp.int32, (SQ, SQ), 0) // BLK
        cb = lax.broadcasted_iota(jnp.int32, (SQ, SQ), 1) // BLK
        mask = cb <= rb
        ctxs = []
        for h in range(H_PER):
            q = Q[:, h * DH:(h + 1) * DH]
            s = lax.dot_general(q, k_ref[h], (((1,), (1,)), ((), ())),
                                preferred_element_type=jnp.float32) * SCALE
            s = jnp.where(mask, s, -1e9)
            m = jnp.max(s, axis=1, keepdims=True)
            w = jnp.exp(s - m)
            w = w / jnp.sum(w, axis=1, keepdims=True)
            ctxs.append(jnp.dot(w, v_ref[h],
                                preferred_element_type=jnp.float32))
        ctx = jnp.concatenate(ctxs, axis=1)
        out_ref[:] = jnp.dot(ctx, wo_ref[:],
                             preferred_element_type=jnp.float32)

        for s_ in range(N_DEV - 1):
            slot = s_ % 2
            if s_ >= 2:
                pl.semaphore_wait(rs_credit, 1)
            send_c = lax.rem(my - s_ + N_DEV, N_DEV)
            recv_c = lax.rem(my - s_ - 1 + N_DEV, N_DEV)
            rdma = pltpu.make_async_remote_copy(
                src_ref=out_ref.at[pl.ds(send_c * CHUNK, CHUNK), :],
                dst_ref=recv_buf.at[slot],
                send_sem=rs_send_sems.at[slot],
                recv_sem=rs_recv_sems.at[slot],
                device_id=(right,),
                device_id_type=pl.DeviceIdType.MESH,
            )
            rdma.start()
            rdma.wait()
            out_ref[pl.ds(recv_c * CHUNK, CHUNK), :] += recv_buf[slot]
            pl.semaphore_signal(rs_credit, inc=1, device_id=(left,),
                                device_id_type=pl.DeviceIdType.MESH)

        for t in range(N_DEV - 1):
            slot = t % 2
            if t >= 2:
                pl.semaphore_wait(ag_credit, 1)
            c = lax.rem(my + 1 - t + N_DEV, N_DEV)
            rdma = pltpu.make_async_remote_copy(
                src_ref=out_ref.at[pl.ds(c * CHUNK, CHUNK), :],
                dst_ref=out_ref.at[pl.ds(c * CHUNK, CHUNK), :],
                send_sem=ag_send_sems.at[slot],
                recv_sem=ag_recv_sems.at[slot],
                device_id=(right,),
                device_id_type=pl.DeviceIdType.MESH,
            )
            rdma.start()
            rdma.wait()
            pl.semaphore_signal(ag_credit, inc=1, device_id=(left,),
                                device_id_type=pl.DeviceIdType.MESH)

    out = pl.pallas_call(
        body,
        out_shape=jax.ShapeDtypeStruct((SQ, D_MODEL), jnp.float32),
        in_specs=[pl.BlockSpec(memory_space=pltpu.VMEM)] * 5,
        out_specs=pl.BlockSpec(memory_space=pltpu.VMEM),
        scratch_shapes=[
            pltpu.VMEM((2, CHUNK, D_MODEL), jnp.float32),
            pltpu.SemaphoreType.DMA((2,)),
            pltpu.SemaphoreType.DMA((2,)),
            pltpu.SemaphoreType.DMA((2,)),
            pltpu.SemaphoreType.DMA((2,)),
            pltpu.SemaphoreType.REGULAR,
            pltpu.SemaphoreType.REGULAR,
        ],
        compiler_params=pltpu.CompilerParams(collective_id=0),
    )(x2, Wq, K, V, Wo)
    return out[None]


# baseline (device time: 176381 ns/iter reference)
import os

import jax
import jax.numpy as jnp
from jax import lax
from jax.experimental import pallas as pl
from jax.experimental.pallas import tpu as pltpu


N_DEV = 16
SQ = 1024
D_MODEL = 1024
H_PER = 8
DH = 128
BLK = 64
CHUNK = SQ // N_DEV
SCALE = 0.08838834764831843

_STAGE = int(os.environ.get("COMM_STAGE", "4"))
_SKIP_COMM = _STAGE == 0
_N_RS = {0: 0, 1: 0, 2: 1, 3: N_DEV - 1, 4: N_DEV - 1}[_STAGE]
_N_AG = {0: 0, 1: 0, 2: 0, 3: 0, 4: N_DEV - 1}[_STAGE]


def kernel(x, Wq, K_ext, V_ext, Wo):
    i = lax.axis_index("i")
    x2 = x[0]
    K = lax.dynamic_slice_in_dim(K_ext[0], i * H_PER, H_PER, axis=1)
    V = lax.dynamic_slice_in_dim(V_ext[0], i * H_PER, H_PER, axis=1)
    K = jnp.transpose(K, (1, 0, 2))
    V = jnp.transpose(V, (1, 0, 2))

    def body(x_ref, wq_ref, k_ref, v_ref, wo_ref, out_ref,
             send_buf, recv_buf, rs_send_sems, rs_recv_sems,
             ag_send_sems, ag_recv_sems, rs_credit, ag_credit):
        my = lax.axis_index("i")
        left = lax.rem(my + N_DEV - 1, N_DEV)
        right = lax.rem(my + 1, N_DEV)

        if not _SKIP_COMM:
            barrier = pltpu.get_barrier_semaphore()
            for nbr in (left, right):
                pl.semaphore_signal(barrier, inc=1, device_id=(nbr,),
                                    device_id_type=pl.DeviceIdType.MESH)
            pl.semaphore_wait(barrier, 2)

        Q = jnp.dot(x_ref[:], wq_ref[:], preferred_element_type=jnp.float32)
        rb = lax.broadcasted_iota(jnp.int32, (SQ, SQ), 0) // BLK
        cb = lax.broadcasted_iota(jnp.int32, (SQ, SQ), 1) // BLK
        mask = cb <= rb
        ctxs = []
        for h in range(H_PER):
            q = Q[:, h * DH:(h + 1) * DH]
            s = lax.dot_general(q, k_ref[h], (((1,), (1,)), ((), ())),
                                preferred_element_type=jnp.float32) * SCALE
            s = jnp.where(mask, s, -1e9)
            m = jnp.max(s, axis=1, keepdims=True)
            w = jnp.exp(s - m)
            w = w / jnp.sum(w, axis=1, keepdims=True)
            ctxs.append(jnp.dot(w, v_ref[h],
                                preferred_element_type=jnp.float32))
        ctx = jnp.concatenate(ctxs, axis=1)
        out_ref[:] = jnp.dot(ctx, wo_ref[:],
                             preferred_element_type=jnp.float32)

        for s_ in range(_N_RS):
            slot = s_ % 2
            if s_ >= 2:
                pl.semaphore_wait(rs_credit, 1)
            send_c = lax.rem(my - s_ + N_DEV, N_DEV)
            recv_c = lax.rem(my - s_ - 1 + N_DEV, N_DEV)
            send_buf[slot] = out_ref[pl.ds(send_c * CHUNK, CHUNK), :]
            rdma = pltpu.make_async_remote_copy(
                src_ref=send_buf.at[slot],
                dst_ref=recv_buf.at[slot],
                send_sem=rs_send_sems.at[slot],
                recv_sem=rs_recv_sems.at[slot],
                device_id=(right,),
                device_id_type=pl.DeviceIdType.MESH,
            )
            rdma.start()
            rdma.wait()
            out_ref[pl.ds(recv_c * CHUNK, CHUNK), :] += recv_buf[slot]
            pl.semaphore_signal(rs_credit, inc=1, device_id=(left,),
                                device_id_type=pl.DeviceIdType.MESH)
        if _N_RS:
            pl.semaphore_wait(rs_credit, min(2, _N_RS))

        for t in range(_N_AG):
            slot = t % 2
            if t >= 2:
                pl.semaphore_wait(ag_credit, 1)
            c = lax.rem(my + 1 - t + N_DEV, N_DEV)
            recv_c = lax.rem(my - t + N_DEV, N_DEV)
            send_buf[slot] = out_ref[pl.ds(c * CHUNK, CHUNK), :]
            rdma = pltpu.make_async_remote_copy(
                src_ref=send_buf.at[slot],
                dst_ref=recv_buf.at[slot],
                send_sem=ag_send_sems.at[slot],
                recv_sem=ag_recv_sems.at[slot],
                device_id=(right,),
                device_id_type=pl.DeviceIdType.MESH,
            )
            rdma.start()
            rdma.wait()
            out_ref[pl.ds(recv_c * CHUNK, CHUNK), :] = recv_buf[slot]
            pl.semaphore_signal(ag_credit, inc=1, device_id=(left,),
                                device_id_type=pl.DeviceIdType.MESH)
        if _N_AG:
            pl.semaphore_wait(ag_credit, min(2, _N_AG))

    out = pl.pallas_call(
        body,
        out_shape=jax.ShapeDtypeStruct((SQ, D_MODEL), jnp.float32),
        in_specs=[pl.BlockSpec(memory_space=pltpu.VMEM)] * 5,
        out_specs=pl.BlockSpec(memory_space=pltpu.VMEM),
        scratch_shapes=[
            pltpu.VMEM((2, CHUNK, D_MODEL), jnp.float32),
            pltpu.VMEM((2, CHUNK, D_MODEL), jnp.float32),
            pltpu.SemaphoreType.DMA((2,)),
            pltpu.SemaphoreType.DMA((2,)),
            pltpu.SemaphoreType.DMA((2,)),
            pltpu.SemaphoreType.DMA((2,)),
            pltpu.SemaphoreType.REGULAR,
            pltpu.SemaphoreType.REGULAR,
        ],
        compiler_params=pltpu.CompilerParams(
            collective_id=None if _SKIP_COMM else 0),
    )(x2, Wq, K, V, Wo)
    return out[None]


# device time: 84008 ns/iter; 2.0996x vs baseline; 2.0996x over previous
import os

import jax
import jax.numpy as jnp
from jax import lax
from jax.experimental import pallas as pl
from jax.experimental.pallas import tpu as pltpu


N_DEV = 16
SQ = 1024
D_MODEL = 1024
H_PER = 8
DH = 128
BLK = 64
CHUNK = SQ // N_DEV
SCALE = 0.08838834764831843

_SKIP_COMM = int(os.environ.get("COMM_STAGE", "4")) == 0


def kernel(x, Wq, K_ext, V_ext, Wo):
    i = lax.axis_index("i")
    x2 = x[0]
    K = lax.dynamic_slice_in_dim(K_ext[0], i * H_PER, H_PER, axis=1)
    V = lax.dynamic_slice_in_dim(V_ext[0], i * H_PER, H_PER, axis=1)
    K = jnp.transpose(K, (1, 0, 2))
    V = jnp.transpose(V, (1, 0, 2))

    def body(x_ref, wq_ref, k_ref, v_ref, wo_ref, out_ref,
             stage_buf, gbuf, gather2, red_buf,
             send1_sems, recv1_sems, send2_sems, recv2_sems):
        my = lax.axis_index("i")

        if not _SKIP_COMM:
            barrier = pltpu.get_barrier_semaphore()
            for k in range(1, N_DEV):
                pl.semaphore_signal(
                    barrier, inc=1,
                    device_id=(lax.rem(my + k, N_DEV),),
                    device_id_type=pl.DeviceIdType.MESH)
            pl.semaphore_wait(barrier, N_DEV - 1)

        bf = jnp.bfloat16
        Q = jnp.dot(x_ref[:].astype(bf), wq_ref[:].astype(bf),
                    preferred_element_type=jnp.float32)
        wo_bf = wo_ref[:].astype(bf)
        QB = 256
        tri = (lax.broadcasted_iota(jnp.int32, (QB, QB), 1) // BLK
               <= lax.broadcasted_iota(jnp.int32, (QB, QB), 0) // BLK)
        for b in range(SQ // QB):
            kv = QB * (b + 1)
            ctxs = []
            for h in range(H_PER):
                q = Q[b * QB:(b + 1) * QB, h * DH:(h + 1) * DH].astype(bf)
                k = k_ref[h, :kv, :].astype(bf)
                s = lax.dot_general(q, k, (((1,), (1,)), ((), ())),
                                    preferred_element_type=jnp.float32
                                    ) * SCALE
                s = jnp.where(
                    jnp.concatenate(
                        [jnp.ones((QB, kv - QB), jnp.bool_), tri], axis=1)
                    if kv > QB else tri,
                    s, -1e9)
                m = jnp.max(s, axis=1, keepdims=True)
                w = jnp.exp(s - m)
                w = (w / jnp.sum(w, axis=1, keepdims=True)).astype(bf)
                ctxs.append(jnp.dot(w, v_ref[h, :kv, :].astype(bf),
                                    preferred_element_type=jnp.float32))
            ctx_b = jnp.concatenate(ctxs, axis=1).astype(bf)
            out_ref[b * QB:(b + 1) * QB, :] = jnp.dot(
                ctx_b, wo_bf, preferred_element_type=jnp.float32)

        if _SKIP_COMM:
            return

        rdmas1 = []
        for j in range(1, N_DEV):
            dest = lax.rem(my + j, N_DEV)
            stage_buf[j - 1] = out_ref[
                pl.ds(dest * CHUNK, CHUNK), :].astype(jnp.bfloat16)
            rdma = pltpu.make_async_remote_copy(
                src_ref=stage_buf.at[j - 1],
                dst_ref=gbuf.at[j - 1],
                send_sem=send1_sems.at[j - 1],
                recv_sem=recv1_sems.at[j - 1],
                device_id=(dest,),
                device_id_type=pl.DeviceIdType.MESH,
            )
            rdma.start()
            rdmas1.append(rdma)

        acc = out_ref[pl.ds(my * CHUNK, CHUNK), :]
        for j in range(1, N_DEV):
            rdmas1[j - 1].wait_recv()
            acc = acc + gbuf[j - 1].astype(jnp.float32)
        red_buf[:] = acc.astype(jnp.bfloat16)
        out_ref[pl.ds(my * CHUNK, CHUNK), :] = acc

        rdmas2 = []
        for j in range(1, N_DEV):
            dest = lax.rem(my + j, N_DEV)
            rdma = pltpu.make_async_remote_copy(
                src_ref=red_buf,
                dst_ref=gather2.at[pl.ds(my * CHUNK, CHUNK), :],
                send_sem=send2_sems.at[j - 1],
                recv_sem=recv2_sems.at[j - 1],
                device_id=(dest,),
                device_id_type=pl.DeviceIdType.MESH,
            )
            rdma.start()
            rdmas2.append(rdma)

        for j in range(1, N_DEV):
            rdmas2[j - 1].wait_recv()
        out_ref[:] = gather2[:].astype(jnp.float32)
        out_ref[pl.ds(my * CHUNK, CHUNK), :] = acc

        for j in range(1, N_DEV):
            rdmas1[j - 1].wait_send()
            rdmas2[j - 1].wait_send()

    out = pl.pallas_call(
        body,
        out_shape=jax.ShapeDtypeStruct((SQ, D_MODEL), jnp.float32),
        in_specs=[pl.BlockSpec(memory_space=pltpu.VMEM)] * 5,
        out_specs=pl.BlockSpec(memory_space=pltpu.VMEM),
        scratch_shapes=[
            pltpu.VMEM((N_DEV - 1, CHUNK, D_MODEL), jnp.bfloat16),
            pltpu.VMEM((N_DEV - 1, CHUNK, D_MODEL), jnp.bfloat16),
            pltpu.VMEM((SQ, D_MODEL), jnp.bfloat16),
            pltpu.VMEM((CHUNK, D_MODEL), jnp.bfloat16),
            pltpu.SemaphoreType.DMA((N_DEV - 1,)),
            pltpu.SemaphoreType.DMA((N_DEV - 1,)),
            pltpu.SemaphoreType.DMA((N_DEV - 1,)),
            pltpu.SemaphoreType.DMA((N_DEV - 1,)),
        ],
        compiler_params=pltpu.CompilerParams(
            collective_id=None if _SKIP_COMM else 0),
    )(x2, Wq, K, V, Wo)
    return out[None]
